# Initial kernel scaffold; baseline (speedup 1.0000x reference)
import jax
import jax.numpy as jnp
from jax.experimental import pallas as pl


def kernel(node_features, node_to_graph_map, alpha, beta, gamma):
    raise NotImplementedError("write your pallas kernel here")


# speedup vs baseline: 10.6076x; 10.6076x over previous
import functools

import jax
import jax.numpy as jnp
from jax.experimental import pallas as pl
from jax.experimental.pallas import tpu as pltpu

N_NODES = 100000
D_FEAT = 128
N_GRAPHS = 64
ROW_BLOCK = 2000
N_BLOCKS = N_NODES // ROW_BLOCK
EPS = 1e-7


def _onehot(ids):
    g = jax.lax.broadcasted_iota(jnp.int32, (ids.shape[0], N_GRAPHS), 1)
    return (ids[:, None] == g).astype(jnp.float32)


def _stats_body(ids_ref, x_ref, a_ref, b_ref, g_ref, scale_ref, bias_ref,
                acc_s, acc_q, acc_c):
    i = pl.program_id(0)

    @pl.when(i == 0)
    def _init():
        acc_s[...] = jnp.zeros_like(acc_s)
        acc_q[...] = jnp.zeros_like(acc_q)
        acc_c[...] = jnp.zeros_like(acc_c)

    ids = ids_ref[0, 0, :]
    oh = _onehot(ids)
    x = x_ref[...]
    dn = (((0,), (0,)), ((), ()))
    acc_s[...] += jax.lax.dot_general(oh, x, dn, preferred_element_type=jnp.float32)
    acc_q[...] += jax.lax.dot_general(oh, x * x, dn, preferred_element_type=jnp.float32)
    ones = jnp.ones((ROW_BLOCK, 1), dtype=jnp.float32)
    acc_c[...] += jax.lax.dot_general(oh, ones, dn, preferred_element_type=jnp.float32)

    @pl.when(i == N_BLOCKS - 1)
    def _finalize():
        cnt = jnp.maximum(acc_c[...], 1.0)
        inv_n = 1.0 / cnt
        mean = acc_s[...] * inv_n
        msq = acc_q[...] * inv_n
        a = a_ref[...]
        var = msq - mean * mean * (2.0 * a - a * a)
        var = jnp.maximum(var, 0.0)
        inv = 1.0 / (jnp.sqrt(var) + EPS)
        scale = inv * g_ref[...]
        scale_ref[...] = scale
        bias_ref[...] = b_ref[...] - a * mean * scale


def _apply_body(ids_ref, x_ref, scale_ref, bias_ref, o_ref):
    ids = ids_ref[0, 0, :]
    oh = _onehot(ids)
    dn = (((1,), (0,)), ((), ()))
    s = jax.lax.dot_general(oh, scale_ref[...], dn, preferred_element_type=jnp.float32)
    b = jax.lax.dot_general(oh, bias_ref[...], dn, preferred_element_type=jnp.float32)
    o_ref[...] = x_ref[...] * s + b


@jax.jit
def kernel(node_features, node_to_graph_map, alpha, beta, gamma):
    ids3 = node_to_graph_map.reshape(N_BLOCKS, 1, ROW_BLOCK)
    a2 = alpha.reshape(1, D_FEAT)
    b2 = beta.reshape(1, D_FEAT)
    g2 = gamma.reshape(1, D_FEAT)

    stats = pl.pallas_call(
        _stats_body,
        grid=(N_BLOCKS,),
        in_specs=[
            pl.BlockSpec((1, 1, ROW_BLOCK), lambda i: (i, 0, 0)),
            pl.BlockSpec((ROW_BLOCK, D_FEAT), lambda i: (i, 0)),
            pl.BlockSpec((1, D_FEAT), lambda i: (0, 0)),
            pl.BlockSpec((1, D_FEAT), lambda i: (0, 0)),
            pl.BlockSpec((1, D_FEAT), lambda i: (0, 0)),
        ],
        out_specs=[
            pl.BlockSpec((N_GRAPHS, D_FEAT), lambda i: (0, 0)),
            pl.BlockSpec((N_GRAPHS, D_FEAT), lambda i: (0, 0)),
        ],
        out_shape=[
            jax.ShapeDtypeStruct((N_GRAPHS, D_FEAT), jnp.float32),
            jax.ShapeDtypeStruct((N_GRAPHS, D_FEAT), jnp.float32),
        ],
        scratch_shapes=[
            pltpu.VMEM((N_GRAPHS, D_FEAT), jnp.float32),
            pltpu.VMEM((N_GRAPHS, D_FEAT), jnp.float32),
            pltpu.VMEM((N_GRAPHS, 1), jnp.float32),
        ],
    )(ids3, node_features, a2, b2, g2)
    scale, bias = stats

    out = pl.pallas_call(
        _apply_body,
        grid=(N_BLOCKS,),
        in_specs=[
            pl.BlockSpec((1, 1, ROW_BLOCK), lambda i: (i, 0, 0)),
            pl.BlockSpec((ROW_BLOCK, D_FEAT), lambda i: (i, 0)),
            pl.BlockSpec((N_GRAPHS, D_FEAT), lambda i: (0, 0)),
            pl.BlockSpec((N_GRAPHS, D_FEAT), lambda i: (0, 0)),
        ],
        out_specs=pl.BlockSpec((ROW_BLOCK, D_FEAT), lambda i: (i, 0)),
        out_shape=jax.ShapeDtypeStruct((N_NODES, D_FEAT), jnp.float32),
    )(ids3, node_features, scale, bias)
    return out
